# initial kernel scaffold (unmeasured)
import jax
import jax.numpy as jnp
from jax import lax
from jax.experimental import pallas as pl
from jax.experimental.pallas import tpu as pltpu

NY, NZ = 4, 4
NREP = NY * NZ
M = 2048
D = 2048
MB = M // NREP


def _snake_pos(y, z):
    return jnp.where(y % 2 == 0, NY * y + z, NY * y + (NZ - 1) - z)


def _snake_coords(p):
    y = p // NZ
    z = jnp.where(y % 2 == 0, p % NZ, (NZ - 1) - (p % NZ))
    return y, z


def _gemm(dy_r, w):
    mb, k = dy_r.shape
    d, _ = w.shape
    bk = 2048
    bn = 512
    kt = k // bk
    nt = d // bn

    def body(dy_ref, w_ref, out_ref):
        ki = pl.program_id(1)

        @pl.when(ki == 0)
        def _():
            out_ref[...] = jnp.zeros_like(out_ref)

        a = dy_ref[...].astype(jnp.bfloat16)
        b = w_ref[...].astype(jnp.bfloat16)
        out_ref[...] += lax.dot_general(
            a, b, (((1,), (1,)), ((), ())),
            preferred_element_type=jnp.float32,
        )

    return pl.pallas_call(
        body,
        grid=(nt, kt),
        in_specs=[
            pl.BlockSpec((mb, bk), lambda i, ki: (0, ki)),
            pl.BlockSpec((bn, bk), lambda i, ki: (i, ki)),
        ],
        out_specs=pl.BlockSpec((mb, bn), lambda i, ki: (0, i)),
        out_shape=jax.ShapeDtypeStruct((mb, d), jnp.float32),
    )(dy_r, w)


def _x_reduce_yz_allgather(partial):

    def body(p_ref, out_ref, xsend, xrecv, ring, sx, rx, ssem, rsem):
        my_x = lax.axis_index("x")
        my_y = lax.axis_index("y")
        my_z = lax.axis_index("z")
        p = _snake_pos(my_y, my_z)
        nxt = (p + 1) % NREP
        ny, nz = _snake_coords(nxt)

        xsend[...] = p_ref[...].astype(jnp.bfloat16)
        rdma_x = pltpu.make_async_remote_copy(
            src_ref=xsend,
            dst_ref=xrecv,
            send_sem=sx,
            recv_sem=rx,
            device_id=(1 - my_x, my_y, my_z),
            device_id_type=pl.DeviceIdType.MESH,
        )
        rdma_x.start()
        rdma_x.wait()

        red = p_ref[...] + xrecv[...].astype(jnp.float32)
        out_ref[pl.ds(p * MB, MB), :] = red
        ring[0] = red.astype(jnp.bfloat16)

        for h in range(NREP - 1):
            s = h % 2
            r = (h + 1) % 2
            rdma = pltpu.make_async_remote_copy(
                src_ref=ring.at[s],
                dst_ref=ring.at[r],
                send_sem=ssem.at[s],
                recv_sem=rsem.at[r],
                device_id=(my_x, ny, nz),
                device_id_type=pl.DeviceIdType.MESH,
            )
            rdma.start()
            rdma.wait()
            origin = (p - h - 1) % NREP
            out_ref[pl.ds(origin * MB, MB), :] = ring[r].astype(jnp.float32)

    return pl.pallas_call(
        body,
        out_shape=jax.ShapeDtypeStruct((M, D), jnp.float32),
        in_specs=[pl.BlockSpec(memory_space=pltpu.VMEM)],
        out_specs=pl.BlockSpec(memory_space=pltpu.VMEM),
        scratch_shapes=[
            pltpu.VMEM((MB, D), jnp.bfloat16),
            pltpu.VMEM((MB, D), jnp.bfloat16),
            pltpu.VMEM((2, MB, D), jnp.bfloat16),
            pltpu.SemaphoreType.DMA,
            pltpu.SemaphoreType.DMA,
            pltpu.SemaphoreType.DMA((2,)),
            pltpu.SemaphoreType.DMA((2,)),
        ],
        compiler_params=pltpu.CompilerParams(collective_id=0),
    )(partial)


def kernel(dy, W):
    my_y = lax.axis_index("y")
    my_z = lax.axis_index("z")
    p = _snake_pos(my_y, my_z)
    dy_r = lax.dynamic_slice(dy, (p * MB, 0), (MB, dy.shape[1]))
    partial = _gemm(dy_r, W)
    return _x_reduce_yz_allgather(partial)


# baseline (device time: 166414 ns/iter reference)
import jax
import jax.numpy as jnp
from jax import lax
from jax.experimental import pallas as pl
from jax.experimental.pallas import tpu as pltpu

NY, NZ = 4, 4
NREP = NY * NZ
M = 2048
D = 2048
MB = M // NREP


def _snake_pos(y, z):
    return jnp.where(y % 2 == 0, NY * y + z, NY * y + (NZ - 1) - z)


def _snake_coords(p):
    y = p // NZ
    z = jnp.where(y % 2 == 0, p % NZ, (NZ - 1) - (p % NZ))
    return y, z


def _gemm(dy_r, w):
    mb, k = dy_r.shape
    d, _ = w.shape
    bk = 2048
    bn = 512
    kt = k // bk
    nt = d // bn

    def body(dy_ref, w_ref, out_ref):
        ki = pl.program_id(1)

        @pl.when(ki == 0)
        def _():
            out_ref[...] = jnp.zeros_like(out_ref)

        a = dy_ref[...].astype(jnp.bfloat16)
        b = w_ref[...].astype(jnp.bfloat16)
        out_ref[...] += lax.dot_general(
            a, b, (((1,), (1,)), ((), ())),
            preferred_element_type=jnp.float32,
        )

    return pl.pallas_call(
        body,
        grid=(nt, kt),
        in_specs=[
            pl.BlockSpec((mb, bk), lambda i, ki: (0, ki)),
            pl.BlockSpec((bn, bk), lambda i, ki: (i, ki)),
        ],
        out_specs=pl.BlockSpec((mb, bn), lambda i, ki: (0, i)),
        out_shape=jax.ShapeDtypeStruct((mb, d), jnp.float32),
    )(dy_r, w)


def _x_reduce_yz_allgather(partial):

    def body(p_ref, out_ref, xsend, xrecv, ring, sx, rx, ssem, rsem):
        my_x = lax.axis_index("x")
        my_y = lax.axis_index("y")
        my_z = lax.axis_index("z")
        p = _snake_pos(my_y, my_z)
        nxt = (p + 1) % NREP
        ny, nz = _snake_coords(nxt)

        xsend[...] = p_ref[...].astype(jnp.bfloat16)
        rdma_x = pltpu.make_async_remote_copy(
            src_ref=xsend,
            dst_ref=xrecv,
            send_sem=sx,
            recv_sem=rx,
            device_id=(1 - my_x, my_y, my_z),
            device_id_type=pl.DeviceIdType.MESH,
        )
        rdma_x.start()
        rdma_x.wait()

        red = p_ref[...] + xrecv[...].astype(jnp.float32)
        out_ref[pl.ds(p * MB, MB), :] = red
        ring[0] = red.astype(jnp.bfloat16)

        for h in range(NREP - 1):
            s = h % 2
            r = (h + 1) % 2
            rdma = pltpu.make_async_remote_copy(
                src_ref=ring.at[s],
                dst_ref=ring.at[r],
                send_sem=ssem.at[s],
                recv_sem=rsem.at[r],
                device_id=(my_x, ny, nz),
                device_id_type=pl.DeviceIdType.MESH,
            )
            rdma.start()
            rdma.wait()
            origin = (p - h - 1) % NREP
            out_ref[pl.ds(origin * MB, MB), :] = ring[r].astype(jnp.float32)

    return pl.pallas_call(
        body,
        out_shape=jax.ShapeDtypeStruct((M, D), jnp.float32),
        in_specs=[pl.BlockSpec(memory_space=pltpu.VMEM)],
        out_specs=pl.BlockSpec(memory_space=pltpu.VMEM),
        scratch_shapes=[
            pltpu.VMEM((MB, D), jnp.bfloat16),
            pltpu.VMEM((MB, D), jnp.bfloat16),
            pltpu.VMEM((2, MB, D), jnp.bfloat16),
            pltpu.SemaphoreType.DMA,
            pltpu.SemaphoreType.DMA,
            pltpu.SemaphoreType.DMA((2,)),
            pltpu.SemaphoreType.DMA((2,)),
        ],
    )(partial)


def kernel(dy, W):
    my_y = lax.axis_index("y")
    my_z = lax.axis_index("z")
    p = _snake_pos(my_y, my_z)
    dy_r = lax.dynamic_slice(dy, (p * MB, 0), (MB, dy.shape[1]))
    partial = _gemm(dy_r, W)
    return _x_reduce_yz_allgather(partial)


# device time: 107645 ns/iter; 1.5460x vs baseline; 1.5460x over previous
import jax
import jax.numpy as jnp
from jax import lax
from jax.experimental import pallas as pl
from jax.experimental.pallas import tpu as pltpu

NY, NZ = 4, 4
NREP = NY * NZ
M = 2048
D = 2048
MB = M // NREP

HAM = [
    (0, 0), (0, 1), (0, 2), (0, 3),
    (1, 3), (1, 2), (1, 1),
    (2, 1), (2, 2), (2, 3),
    (3, 3), (3, 2), (3, 1), (3, 0),
    (2, 0), (1, 0),
]
assert len(HAM) == NREP and len(set(HAM)) == NREP
for _a, _b in zip(HAM, HAM[1:] + HAM[:1]):
    assert abs(_a[0] - _b[0]) + abs(_a[1] - _b[1]) == 1, (_a, _b)

N_CW = NREP // 2
N_CCW = NREP - 1 - N_CW


def _ring_pos(y, z):
    p = jnp.int32(0)
    for i, (yy, zz) in enumerate(HAM):
        p = jnp.where((y == yy) & (z == zz), jnp.int32(i), p)
    return p


def _coords_at(p, offset):
    ny = jnp.int32(0)
    nz = jnp.int32(0)
    for i in range(NREP):
        yy, zz = HAM[(i + offset) % NREP]
        ny = jnp.where(p == i, jnp.int32(yy), ny)
        nz = jnp.where(p == i, jnp.int32(zz), nz)
    return ny, nz


def _gemm(dy_r, w):
    mb, k = dy_r.shape
    d, _ = w.shape
    bk = 2048
    bn = 512
    kt = k // bk
    nt = d // bn

    def body(dy_ref, w_ref, out_ref):
        ki = pl.program_id(1)

        @pl.when(ki == 0)
        def _():
            out_ref[...] = jnp.zeros_like(out_ref)

        a = dy_ref[...].astype(jnp.bfloat16)
        b = w_ref[...].astype(jnp.bfloat16)
        out_ref[...] += lax.dot_general(
            a, b, (((1,), (1,)), ((), ())),
            preferred_element_type=jnp.float32,
        )

    return pl.pallas_call(
        body,
        grid=(nt, kt),
        in_specs=[
            pl.BlockSpec((mb, bk), lambda i, ki: (0, ki)),
            pl.BlockSpec((bn, bk), lambda i, ki: (i, ki)),
        ],
        out_specs=pl.BlockSpec((mb, bn), lambda i, ki: (0, i)),
        out_shape=jax.ShapeDtypeStruct((mb, d), jnp.float32),
    )(dy_r, w)


def _x_reduce_yz_allgather(partial):

    def body(p_ref, out_ref, xsend, xrecv, cw, ccw,
             sx, rx, cw_s, cw_r, ccw_s, ccw_r):
        my_x = lax.axis_index("x")
        my_y = lax.axis_index("y")
        my_z = lax.axis_index("z")
        p = _ring_pos(my_y, my_z)
        ny1, nz1 = _coords_at(p, 1)
        py1, pz1 = _coords_at(p, -1)
        nxt = (my_x, ny1, nz1)
        prv = (my_x, py1, pz1)
        xpartner = (1 - my_x, my_y, my_z)

        barrier = pltpu.get_barrier_semaphore()
        for dev in (nxt, prv, xpartner):
            pl.semaphore_signal(
                barrier, inc=1, device_id=dev,
                device_id_type=pl.DeviceIdType.MESH,
            )
        pl.semaphore_wait(barrier, 3)

        xsend[...] = p_ref[...].astype(jnp.bfloat16)
        rdma_x = pltpu.make_async_remote_copy(
            src_ref=xsend,
            dst_ref=xrecv,
            send_sem=sx,
            recv_sem=rx,
            device_id=xpartner,
            device_id_type=pl.DeviceIdType.MESH,
        )
        rdma_x.start()
        rdma_x.wait()

        red = p_ref[...] + xrecv[...].astype(jnp.float32)
        out_ref[pl.ds(p * MB, MB), :] = red
        red16 = red.astype(jnp.bfloat16)
        cw[0] = red16
        ccw[0] = red16

        for h in range(N_CW):
            s, r = h % 2, (h + 1) % 2
            rc = pltpu.make_async_remote_copy(
                src_ref=cw.at[s], dst_ref=cw.at[r],
                send_sem=cw_s.at[s], recv_sem=cw_r.at[r],
                device_id=nxt,
                device_id_type=pl.DeviceIdType.MESH,
            )
            rc.start()
            if h < N_CCW:
                rcc = pltpu.make_async_remote_copy(
                    src_ref=ccw.at[s], dst_ref=ccw.at[r],
                    send_sem=ccw_s.at[s], recv_sem=ccw_r.at[r],
                    device_id=prv,
                    device_id_type=pl.DeviceIdType.MESH,
                )
                rcc.start()
            if h > 0:
                out_ref[pl.ds(((p - h) % NREP) * MB, MB), :] = (
                    cw[s].astype(jnp.float32))
                out_ref[pl.ds(((p + h) % NREP) * MB, MB), :] = (
                    ccw[s].astype(jnp.float32))
            rc.wait()
            if h < N_CCW:
                rcc.wait()
        out_ref[pl.ds(((p - N_CW) % NREP) * MB, MB), :] = (
            cw[N_CW % 2].astype(jnp.float32))

    return pl.pallas_call(
        body,
        out_shape=jax.ShapeDtypeStruct((M, D), jnp.float32),
        in_specs=[pl.BlockSpec(memory_space=pltpu.VMEM)],
        out_specs=pl.BlockSpec(memory_space=pltpu.VMEM),
        scratch_shapes=[
            pltpu.VMEM((MB, D), jnp.bfloat16),
            pltpu.VMEM((MB, D), jnp.bfloat16),
            pltpu.VMEM((2, MB, D), jnp.bfloat16),
            pltpu.VMEM((2, MB, D), jnp.bfloat16),
            pltpu.SemaphoreType.DMA,
            pltpu.SemaphoreType.DMA,
            pltpu.SemaphoreType.DMA((2,)),
            pltpu.SemaphoreType.DMA((2,)),
            pltpu.SemaphoreType.DMA((2,)),
            pltpu.SemaphoreType.DMA((2,)),
        ],
        compiler_params=pltpu.CompilerParams(collective_id=0),
    )(partial)


def kernel(dy, W):
    my_y = lax.axis_index("y")
    my_z = lax.axis_index("z")
    p = _ring_pos(my_y, my_z)
    dy_r = lax.dynamic_slice(dy, (p * MB, 0), (MB, dy.shape[1]))
    partial = _gemm(dy_r, W)
    return _x_reduce_yz_allgather(partial)


# device time: 90402 ns/iter; 1.8408x vs baseline; 1.1907x over previous
import jax
import jax.numpy as jnp
from jax import lax
from jax.experimental import pallas as pl
from jax.experimental.pallas import tpu as pltpu

NY, NZ = 4, 4
NREP = NY * NZ
M = 2048
D = 2048
TM = 512
TN = 512
TGRID = D // TN

HAM = [
    (0, 0), (0, 1), (0, 2), (0, 3),
    (1, 3), (1, 2), (1, 1),
    (2, 1), (2, 2), (2, 3),
    (3, 3), (3, 2), (3, 1), (3, 0),
    (2, 0), (1, 0),
]
assert len(HAM) == NREP and len(set(HAM)) == NREP
for _a, _b in zip(HAM, HAM[1:] + HAM[:1]):
    assert abs(_a[0] - _b[0]) + abs(_a[1] - _b[1]) == 1, (_a, _b)

N_CW = NREP // 2
N_CCW = NREP - 1 - N_CW


def _ring_pos(y, z):
    p = jnp.int32(0)
    for i, (yy, zz) in enumerate(HAM):
        p = jnp.where((y == yy) & (z == zz), jnp.int32(i), p)
    return p


def _coords_at(p, offset):
    ny = jnp.int32(0)
    nz = jnp.int32(0)
    for i in range(NREP):
        yy, zz = HAM[(i + offset) % NREP]
        ny = jnp.where(p == i, jnp.int32(yy), ny)
        nz = jnp.where(p == i, jnp.int32(zz), nz)
    return ny, nz


def _gemm_tile(tile_idx, dy, w):
    _, k = dy.shape
    bk = 2048
    kt = k // bk

    def body(idx_ref, dy_ref, w_ref, out_ref):
        ki = pl.program_id(0)

        @pl.when(ki == 0)
        def _():
            out_ref[...] = jnp.zeros_like(out_ref)

        a = dy_ref[...].astype(jnp.bfloat16)
        b = w_ref[...].astype(jnp.bfloat16)
        out_ref[...] += lax.dot_general(
            a, b, (((1,), (1,)), ((), ())),
            preferred_element_type=jnp.float32,
        )

    grid_spec = pltpu.PrefetchScalarGridSpec(
        num_scalar_prefetch=1,
        grid=(kt,),
        in_specs=[
            pl.BlockSpec((TM, bk), lambda ki, idx: (idx[0], ki)),
            pl.BlockSpec((TN, bk), lambda ki, idx: (idx[1], ki)),
        ],
        out_specs=pl.BlockSpec((TM, TN), lambda ki, idx: (0, 0)),
    )
    return pl.pallas_call(
        body,
        grid_spec=grid_spec,
        out_shape=jax.ShapeDtypeStruct((TM, TN), jnp.float32),
    )(tile_idx, dy, w)


def _x_reduce_yz_allgather(partial):

    def _store(out_ref, o, block16):
        orow = (o // TGRID) * TM
        ocol = (o % TGRID) * TN
        out_ref[pl.ds(orow, TM), pl.ds(ocol, TN)] = block16.astype(jnp.float32)

    def body(p_ref, out_ref, xsend, xrecv, cw, ccw,
             sx, rx, cw_s, cw_r, ccw_s, ccw_r):
        my_x = lax.axis_index("x")
        my_y = lax.axis_index("y")
        my_z = lax.axis_index("z")
        p = _ring_pos(my_y, my_z)
        ny1, nz1 = _coords_at(p, 1)
        py1, pz1 = _coords_at(p, -1)
        nxt = (my_x, ny1, nz1)
        prv = (my_x, py1, pz1)
        xpartner = (1 - my_x, my_y, my_z)

        barrier = pltpu.get_barrier_semaphore()
        for dev in (nxt, prv, xpartner):
            pl.semaphore_signal(
                barrier, inc=1, device_id=dev,
                device_id_type=pl.DeviceIdType.MESH,
            )
        pl.semaphore_wait(barrier, 3)

        xsend[...] = p_ref[...].astype(jnp.bfloat16)
        rdma_x = pltpu.make_async_remote_copy(
            src_ref=xsend,
            dst_ref=xrecv,
            send_sem=sx,
            recv_sem=rx,
            device_id=xpartner,
            device_id_type=pl.DeviceIdType.MESH,
        )
        rdma_x.start()
        rdma_x.wait()

        red = p_ref[...] + xrecv[...].astype(jnp.float32)
        red16 = red.astype(jnp.bfloat16)
        orow = (p // TGRID) * TM
        ocol = (p % TGRID) * TN
        out_ref[pl.ds(orow, TM), pl.ds(ocol, TN)] = red
        cw[0] = red16
        ccw[0] = red16

        for h in range(N_CW):
            s, r = h % 2, (h + 1) % 2
            rc = pltpu.make_async_remote_copy(
                src_ref=cw.at[s], dst_ref=cw.at[r],
                send_sem=cw_s.at[s], recv_sem=cw_r.at[r],
                device_id=nxt,
                device_id_type=pl.DeviceIdType.MESH,
            )
            rc.start()
            if h < N_CCW:
                rcc = pltpu.make_async_remote_copy(
                    src_ref=ccw.at[s], dst_ref=ccw.at[r],
                    send_sem=ccw_s.at[s], recv_sem=ccw_r.at[r],
                    device_id=prv,
                    device_id_type=pl.DeviceIdType.MESH,
                )
                rcc.start()
            if h > 0:
                _store(out_ref, (p - h) % NREP, cw[s])
                _store(out_ref, (p + h) % NREP, ccw[s])
            rc.wait()
            if h < N_CCW:
                rcc.wait()
        _store(out_ref, (p - N_CW) % NREP, cw[N_CW % 2])

    return pl.pallas_call(
        body,
        out_shape=jax.ShapeDtypeStruct((M, D), jnp.float32),
        in_specs=[pl.BlockSpec(memory_space=pltpu.VMEM)],
        out_specs=pl.BlockSpec(memory_space=pltpu.VMEM),
        scratch_shapes=[
            pltpu.VMEM((TM, TN), jnp.bfloat16),
            pltpu.VMEM((TM, TN), jnp.bfloat16),
            pltpu.VMEM((2, TM, TN), jnp.bfloat16),
            pltpu.VMEM((2, TM, TN), jnp.bfloat16),
            pltpu.SemaphoreType.DMA,
            pltpu.SemaphoreType.DMA,
            pltpu.SemaphoreType.DMA((2,)),
            pltpu.SemaphoreType.DMA((2,)),
            pltpu.SemaphoreType.DMA((2,)),
            pltpu.SemaphoreType.DMA((2,)),
        ],
        compiler_params=pltpu.CompilerParams(collective_id=0),
    )(partial)


def kernel(dy, W):
    my_y = lax.axis_index("y")
    my_z = lax.axis_index("z")
    p = _ring_pos(my_y, my_z)
    tile_idx = jnp.stack([p // TGRID, p % TGRID]).astype(jnp.int32)
    partial = _gemm_tile(tile_idx, dy, W)
    return _x_reduce_yz_allgather(partial)


# device time: 74289 ns/iter; 2.2401x vs baseline; 1.2169x over previous
import jax
import jax.numpy as jnp
from jax import lax
from jax.experimental import pallas as pl
from jax.experimental.pallas import tpu as pltpu

NY, NZ = 4, 4
NREP = NY * NZ
M = 2048
D = 2048
TM = 512
TN = 512
TGRID = D // TN

HAM = [
    (0, 0), (0, 1), (0, 2), (0, 3),
    (1, 3), (1, 2), (1, 1),
    (2, 1), (2, 2), (2, 3),
    (3, 3), (3, 2), (3, 1), (3, 0),
    (2, 0), (1, 0),
]
assert len(HAM) == NREP and len(set(HAM)) == NREP
for _a, _b in zip(HAM, HAM[1:] + HAM[:1]):
    assert abs(_a[0] - _b[0]) + abs(_a[1] - _b[1]) == 1, (_a, _b)

N_CW = NREP // 2
N_CCW = NREP - 1 - N_CW


def _ring_pos(y, z):
    p = jnp.int32(0)
    for i, (yy, zz) in enumerate(HAM):
        p = jnp.where((y == yy) & (z == zz), jnp.int32(i), p)
    return p


def _coords_at(p, offset):
    ny = jnp.int32(0)
    nz = jnp.int32(0)
    for i in range(NREP):
        yy, zz = HAM[(i + offset) % NREP]
        ny = jnp.where(p == i, jnp.int32(yy), ny)
        nz = jnp.where(p == i, jnp.int32(zz), nz)
    return ny, nz


def _gemm_tile(tile_idx, dy, w):
    _, k = dy.shape
    bk = 2048
    kt = k // bk

    def body(idx_ref, dy_ref, w_ref, out_ref):
        ki = pl.program_id(0)

        @pl.when(ki == 0)
        def _():
            out_ref[...] = jnp.zeros_like(out_ref)

        a = dy_ref[...].astype(jnp.bfloat16)
        b = w_ref[...].astype(jnp.bfloat16)
        out_ref[...] += lax.dot_general(
            a, b, (((1,), (1,)), ((), ())),
            preferred_element_type=jnp.float32,
        )

    grid_spec = pltpu.PrefetchScalarGridSpec(
        num_scalar_prefetch=1,
        grid=(kt,),
        in_specs=[
            pl.BlockSpec((TM, bk), lambda ki, idx: (idx[0], ki)),
            pl.BlockSpec((TN, bk), lambda ki, idx: (idx[1], ki)),
        ],
        out_specs=pl.BlockSpec((TM, TN), lambda ki, idx: (0, 0)),
    )
    return pl.pallas_call(
        body,
        grid_spec=grid_spec,
        out_shape=jax.ShapeDtypeStruct((TM, TN), jnp.float32),
    )(tile_idx, dy, w)


HM = TM // 2


def _x_reduce_yz_allgather(partial):

    def _store_half(out_ref, o, v, block16):
        orow = (o // TGRID) * TM + v * HM
        ocol = (o % TGRID) * TN
        out_ref[pl.ds(orow, HM), pl.ds(ocol, TN)] = block16.astype(jnp.float32)

    def body(p_ref, out_ref, xsend, xrecv, cw, ccw,
             sx, rx, cw_s, cw_r, ccw_s, ccw_r):
        my_x = lax.axis_index("x")
        my_y = lax.axis_index("y")
        my_z = lax.axis_index("z")
        p = _ring_pos(my_y, my_z)
        ny1, nz1 = _coords_at(p, 1)
        py1, pz1 = _coords_at(p, -1)
        nxt = (my_x, ny1, nz1)
        prv = (my_x, py1, pz1)
        xpartner = (1 - my_x, my_y, my_z)

        barrier = pltpu.get_barrier_semaphore()
        for dev in (nxt, prv, xpartner):
            pl.semaphore_signal(
                barrier, inc=1, device_id=dev,
                device_id_type=pl.DeviceIdType.MESH,
            )
        pl.semaphore_wait(barrier, 3)

        def ring_rdma(bufs, sems_s, sems_r, dev, v, h):
            s, r = h % 2, (h + 1) % 2
            return pltpu.make_async_remote_copy(
                src_ref=bufs.at[v, s], dst_ref=bufs.at[v, r],
                send_sem=sems_s.at[v, s], recv_sem=sems_r.at[v, r],
                device_id=dev, device_id_type=pl.DeviceIdType.MESH,
            )

        xsend[0] = p_ref[0:HM, :].astype(jnp.bfloat16)
        xsend[1] = p_ref[HM:TM, :].astype(jnp.bfloat16)
        xr = [
            pltpu.make_async_remote_copy(
                src_ref=xsend.at[v], dst_ref=xrecv.at[v],
                send_sem=sx.at[v], recv_sem=rx.at[v],
                device_id=xpartner, device_id_type=pl.DeviceIdType.MESH,
            )
            for v in range(2)
        ]
        xr[0].start()
        xr[1].start()

        hop0 = [None, None]
        for v in range(2):
            xr[v].wait()
            red = p_ref[pl.ds(v * HM, HM), :] + xrecv[v].astype(jnp.float32)
            red16 = red.astype(jnp.bfloat16)
            cw[v, 0] = red16
            ccw[v, 0] = red16
            rc = ring_rdma(cw, cw_s, cw_r, nxt, v, 0)
            rcc = ring_rdma(ccw, ccw_s, ccw_r, prv, v, 0)
            rc.start()
            rcc.start()
            hop0[v] = (rc, rcc)
            _store_half(out_ref, p, v, red)

        pending = {0: hop0[0], 1: hop0[1]}
        for h in range(1, N_CW):
            for v in range(2):
                rc_prev, rcc_prev = pending[v]
                rc_prev.wait()
                if h - 1 < N_CCW:
                    rcc_prev.wait()
                rc = ring_rdma(cw, cw_s, cw_r, nxt, v, h)
                rc.start()
                rcc = None
                if h < N_CCW:
                    rcc = ring_rdma(ccw, ccw_s, ccw_r, prv, v, h)
                    rcc.start()
                pending[v] = (rc, rcc)
                _store_half(out_ref, (p - h) % NREP, v, cw[v, h % 2])
                _store_half(out_ref, (p + h) % NREP, v, ccw[v, h % 2])
        for v in range(2):
            rc_prev, rcc_prev = pending[v]
            rc_prev.wait()
            if N_CW - 1 < N_CCW:
                rcc_prev.wait()
            _store_half(out_ref, (p - N_CW) % NREP, v, cw[v, N_CW % 2])

    return pl.pallas_call(
        body,
        out_shape=jax.ShapeDtypeStruct((M, D), jnp.float32),
        in_specs=[pl.BlockSpec(memory_space=pltpu.VMEM)],
        out_specs=pl.BlockSpec(memory_space=pltpu.VMEM),
        scratch_shapes=[
            pltpu.VMEM((2, HM, TN), jnp.bfloat16),
            pltpu.VMEM((2, HM, TN), jnp.bfloat16),
            pltpu.VMEM((2, 2, HM, TN), jnp.bfloat16),
            pltpu.VMEM((2, 2, HM, TN), jnp.bfloat16),
            pltpu.SemaphoreType.DMA((2,)),
            pltpu.SemaphoreType.DMA((2,)),
            pltpu.SemaphoreType.DMA((2, 2)),
            pltpu.SemaphoreType.DMA((2, 2)),
            pltpu.SemaphoreType.DMA((2, 2)),
            pltpu.SemaphoreType.DMA((2, 2)),
        ],
        compiler_params=pltpu.CompilerParams(collective_id=0),
    )(partial)


def kernel(dy, W):
    my_y = lax.axis_index("y")
    my_z = lax.axis_index("z")
    p = _ring_pos(my_y, my_z)
    tile_idx = jnp.stack([p // TGRID, p % TGRID]).astype(jnp.int32)
    partial = _gemm_tile(tile_idx, dy, W)
    return _x_reduce_yz_allgather(partial)
